# baseline (speedup 1.0000x reference)
import functools

import jax
import jax.numpy as jnp
from jax import lax
from jax.experimental import pallas as pl
from jax.experimental.pallas import tpu as pltpu
from jax.experimental.pallas import tpu_sc as plsc

B = 8
M = 65536
D = 64
SCALE = 2.5
SLOPE = 0.01

MSC = 32768
MS = M - MSC

SEG = 4
NW = 32
KEYS = max(MSC // SEG, 1024)
CH = 256
NCH = KEYS // CH
GR = CH // 16

CHT = 4096
NCT = MS // CHT

_LOG2E = 1.4426950408889634
_LN2 = 0.6931471805599453
_RND = 12582912.0


def _bf16_rne(x):
    i = lax.bitcast_convert_type(x, jnp.int32)
    lsb = lax.shift_right_logical(i, 16) & 1
    r = (i + 0x7FFF + lsb) & jnp.int32(-65536)
    return lax.bitcast_convert_type(r, jnp.float32)


def _exp_precise(x):
    x = jnp.maximum(x, -87.0)
    t = x * _LOG2E
    n_f = (t + _RND) - _RND
    r = (t - n_f) * _LN2
    p = 1.0 + r * (1.0 + r * (0.5 + r * (1.0 / 6.0 + r * (
        1.0 / 24.0 + r * (1.0 / 120.0 + r * (1.0 / 720.0))))))
    n_i = n_f.astype(jnp.int32)
    scale = lax.bitcast_convert_type((n_i + 127) << 23, jnp.float32)
    return p * scale


_mesh = plsc.VectorSubcoreMesh(core_axis_name="c", subcore_axis_name="s")


@functools.partial(
    pl.kernel,
    mesh=_mesh,
    compiler_params=pltpu.CompilerParams(needs_layout_passes=False),
    out_type=[
        jax.ShapeDtypeStruct((NW * 64,), jnp.float32),
        jax.ShapeDtypeStruct((NW * 32,), jnp.float32),
    ],
    scratch_types=[
        pltpu.VMEM((D, CH), jnp.float32),
        pltpu.VMEM((D, CH), jnp.float32),
        pltpu.VMEM((KEYS,), jnp.float32),
        pltpu.VMEM((KEYS,), jnp.int32),
        pltpu.VMEM((64,), jnp.float32),
        pltpu.VMEM((16,), jnp.float32),
        pltpu.VMEM((1024,), jnp.float32),
        pltpu.VMEM((1024,), jnp.float32),
        pltpu.VMEM((64,), jnp.float32),
        pltpu.VMEM((32,), jnp.float32),
        pltpu.SemaphoreType.DMA,
        pltpu.SemaphoreType.DMA,
    ],
)
def _sc_attn(q_hbm, kt_hbm, vt_hbm, pos_hbm, qp_hbm, acc_out, st_out,
             buf0, buf1, scores, posb, qv, qpb, qb, accv, stacc, ststat,
             sem0, sem1):
    cid = lax.axis_index("c")
    sid = lax.axis_index("s")
    wid = cid * 16 + sid
    b = wid // SEG
    seg = wid % SEG
    base = MS + seg * KEYS

    pltpu.sync_copy(q_hbm.at[b], qv)
    pltpu.sync_copy(qp_hbm, qpb)
    pltpu.sync_copy(pos_hbm.at[b, pl.ds(base, KEYS)], posb)

    lane = lax.iota(jnp.int32, 16)
    zero = jnp.zeros((16,), jnp.float32)
    qp_vec = qpb[...]

    for i in range(4):
        qvec = _bf16_rne(qv[pl.ds(16 * i, 16)])
        for j in range(16):
            qb[pl.ds((16 * i + j) * 16, 16)] = zero + qvec[j]
            accv[pl.ds((16 * i + j) * 16, 16)] = zero

    def kcopy(c, buf, sem):
        return pltpu.make_async_copy(
            kt_hbm.at[b, :, pl.ds(base + c * CH, CH)], buf, sem)

    def vcopy(c, buf, sem):
        return pltpu.make_async_copy(
            vt_hbm.at[b, :, pl.ds(base + c * CH, CH)], buf, sem)

    def p1_chunk(c, buf, m_run):
        def block(bi, m_run):
            k0 = bi * 64

            def dloop(d, svs):
                s0, s1, s2, s3 = svs
                qd = qb[pl.ds(d * 16, 16)]
                s0 = s0 + qd * _bf16_rne(buf[d, pl.ds(k0, 16)])
                s1 = s1 + qd * _bf16_rne(buf[d, pl.ds(k0 + 16, 16)])
                s2 = s2 + qd * _bf16_rne(buf[d, pl.ds(k0 + 32, 16)])
                s3 = s3 + qd * _bf16_rne(buf[d, pl.ds(k0 + 48, 16)])
                return (s0, s1, s2, s3)

            svs = lax.fori_loop(0, 64, dloop, (zero, zero, zero, zero))
            kk = c * CH + k0
            for t in range(4):
                pos16 = posb[pl.ds(kk + t * 16, 16)].astype(jnp.float32)
                sv = svs[t] * SCALE - SLOPE * jnp.abs(qp_vec - pos16)
                scores[pl.ds(kk + t * 16, 16)] = sv
                m_run = jnp.maximum(m_run, sv)
            return m_run
        return lax.fori_loop(0, CH // 64, block, m_run)

    kcopy(0, buf0, sem0).start()

    def outer1(i, m_run):
        c0 = 2 * i
        kcopy(c0, buf0, sem0).wait()
        kcopy(c0 + 1, buf1, sem1).start()
        m_run = p1_chunk(c0, buf0, m_run)
        kcopy(c0 + 1, buf1, sem1).wait()

        @pl.when(i < NCH // 2 - 1)
        def _():
            kcopy(c0 + 2, buf0, sem0).start()

        m_run = p1_chunk(c0 + 1, buf1, m_run)
        return m_run

    m_run = lax.fori_loop(0, NCH // 2, outer1,
                          jnp.full((16,), -3e38, jnp.float32))
    gmax = jnp.max(m_run)

    def p2_chunk(c, buf, ws):
        def block(bi, ws):
            k0 = bi * 64
            kk = c * CH + k0
            w0 = _exp_precise(scores[pl.ds(kk, 16)] - gmax)
            w1 = _exp_precise(scores[pl.ds(kk + 16, 16)] - gmax)
            w2 = _exp_precise(scores[pl.ds(kk + 32, 16)] - gmax)
            w3 = _exp_precise(scores[pl.ds(kk + 48, 16)] - gmax)
            ws = ws + ((w0 + w1) + (w2 + w3))

            def dloop(d, t):
                acc = accv[pl.ds(d * 16, 16)]
                acc = acc + w0 * buf[d, pl.ds(k0, 16)]
                acc = acc + w1 * buf[d, pl.ds(k0 + 16, 16)]
                acc = acc + w2 * buf[d, pl.ds(k0 + 32, 16)]
                acc = acc + w3 * buf[d, pl.ds(k0 + 48, 16)]
                accv[pl.ds(d * 16, 16)] = acc
                return t

            lax.fori_loop(0, 64, dloop, 0)
            return ws
        return lax.fori_loop(0, CH // 64, block, ws)

    vcopy(0, buf0, sem0).start()

    def outer2(i, ws):
        c0 = 2 * i
        vcopy(c0, buf0, sem0).wait()
        vcopy(c0 + 1, buf1, sem1).start()
        ws = p2_chunk(c0, buf0, ws)
        vcopy(c0 + 1, buf1, sem1).wait()

        @pl.when(i < NCH // 2 - 1)
        def _():
            vcopy(c0 + 2, buf0, sem0).start()

        ws = p2_chunk(c0 + 1, buf1, ws)
        return ws

    ws = lax.fori_loop(0, NCH // 2, outer2, zero)

    for t in range(4):
        vec = zero
        for j in range(16):
            d = t * 16 + j
            vec = jnp.where(lane == j, jnp.sum(accv[pl.ds(d * 16, 16)]), vec)
        stacc[pl.ds(t * 16, 16)] = vec
    pltpu.sync_copy(stacc, acc_out.at[pl.ds(wid * 64, 64)])

    ststat[pl.ds(0, 16)] = zero + gmax
    ststat[pl.ds(16, 16)] = ws
    pltpu.sync_copy(ststat, st_out.at[pl.ds(wid * 32, 32)])


def _tc_body(q_ref, kt_ref, vt_ref, pos_ref, qp_ref, acc_out, st_out):
    qp = qp_ref[0]
    q16 = q_ref[0].astype(jnp.bfloat16)
    k16 = kt_ref[0].astype(jnp.bfloat16)
    s = lax.dot_general(q16, k16, (((1,), (0,)), ((), ())),
                        preferred_element_type=jnp.float32)
    pos = pos_ref[0].astype(jnp.float32)
    s = s * SCALE - SLOPE * jnp.abs(qp - pos)
    m_c = jnp.max(s)
    p = jnp.exp(s - m_c)
    l_c = jnp.sum(p)
    pv = lax.dot_general(p.astype(jnp.bfloat16),
                         vt_ref[0].astype(jnp.bfloat16),
                         (((1,), (1,)), ((), ())),
                         preferred_element_type=jnp.float32)
    acc_out[...] = pv.reshape(1, 1, 1, 64)
    iota = lax.broadcasted_iota(jnp.int32, (1, 1, 1, 128), 3)
    st_out[...] = jnp.where(iota == 0, m_c,
                            jnp.where(iota == 1, l_c, 0.0))


_tc_attn = pl.pallas_call(
    _tc_body,
    grid=(B, NCT if NCT else 1),
    in_specs=[
        pl.BlockSpec((1, 1, 64), lambda b, c: (b, 0, 0)),
        pl.BlockSpec((1, 64, CHT), lambda b, c: (b, 0, c)),
        pl.BlockSpec((1, 64, CHT), lambda b, c: (b, 0, c)),
        pl.BlockSpec((1, 1, CHT), lambda b, c: (b * (NCT if NCT else 1) + c, 0, 0)),
        pl.BlockSpec(memory_space=pltpu.MemorySpace.SMEM),
    ],
    out_specs=[
        pl.BlockSpec((1, 1, 1, 64), lambda b, c: (b, c, 0, 0)),
        pl.BlockSpec((1, 1, 1, 128), lambda b, c: (b, c, 0, 0)),
    ],
    out_shape=[
        jax.ShapeDtypeStruct((B, NCT if NCT else 1, 1, 64), jnp.float32),
        jax.ShapeDtypeStruct((B, NCT if NCT else 1, 1, 128), jnp.float32),
    ],
)


def kernel(query_addr, key_addrs, values, query_pos, key_pos):
    pos32 = key_pos.astype(jnp.int32)
    qp_s = jnp.full((1,), query_pos, dtype=jnp.float32)
    kt = jnp.swapaxes(key_addrs, 1, 2)
    vt = jnp.swapaxes(values, 1, 2)

    parts_m = []
    parts_l = []
    parts_acc = []

    if MSC:
        qp = jnp.full((16,), query_pos, dtype=jnp.float32)
        accf, stf = _sc_attn(query_addr, kt, vt, pos32, qp)
        acc = accf.reshape(B, SEG, 64)
        st = stf.reshape(B, SEG, 32)
        parts_m.append(st[:, :, 0])
        parts_l.append(st[:, :, 16:32].sum(-1))
        parts_acc.append(acc)

    if NCT:
        q3d = query_addr.reshape(B, 1, 64)
        pos_tc = pos32[:, :MS].reshape(B * NCT, 1, CHT)
        tacc, tst = _tc_attn(q3d, kt, vt, pos_tc, qp_s)
        parts_m.append(tst[:, :, 0, 0])
        parts_l.append(tst[:, :, 0, 1])
        parts_acc.append(tacc[:, :, 0, :])

    pm = jnp.concatenate(parts_m, axis=1)
    pl_ = jnp.concatenate(parts_l, axis=1)
    pa = jnp.concatenate(parts_acc, axis=1)
    gm = pm.max(axis=1, keepdims=True)
    sc = jnp.exp(pm - gm)
    num = (sc[:, :, None] * pa).sum(1)
    den = (sc * pl_).sum(1)[:, None]
    return num / den


# speedup vs baseline: 1.2660x; 1.2660x over previous
import functools

import jax
import jax.numpy as jnp
from jax import lax
from jax.experimental import pallas as pl
from jax.experimental.pallas import tpu as pltpu
from jax.experimental.pallas import tpu_sc as plsc

B = 8
M = 65536
D = 64
SCALE = 2.5
SLOPE = 0.01

MSC = 24576
MS = M - MSC

SEG = 4
NW = 32
KEYS = max(MSC // SEG, 1024)
CH = 256
NCH = KEYS // CH
GR = CH // 16

CHT = 4096
NCT = MS // CHT

_LOG2E = 1.4426950408889634
_LN2 = 0.6931471805599453
_RND = 12582912.0


def _bf16_rne(x):
    i = lax.bitcast_convert_type(x, jnp.int32)
    lsb = lax.shift_right_logical(i, 16) & 1
    r = (i + 0x7FFF + lsb) & jnp.int32(-65536)
    return lax.bitcast_convert_type(r, jnp.float32)


def _exp_precise(x):
    x = jnp.maximum(x, -87.0)
    t = x * _LOG2E
    n_f = (t + _RND) - _RND
    r = (t - n_f) * _LN2
    p = 1.0 + r * (1.0 + r * (0.5 + r * (1.0 / 6.0 + r * (
        1.0 / 24.0 + r * (1.0 / 120.0 + r * (1.0 / 720.0))))))
    n_i = n_f.astype(jnp.int32)
    scale = lax.bitcast_convert_type((n_i + 127) << 23, jnp.float32)
    return p * scale


_mesh = plsc.VectorSubcoreMesh(core_axis_name="c", subcore_axis_name="s")


@functools.partial(
    pl.kernel,
    mesh=_mesh,
    compiler_params=pltpu.CompilerParams(needs_layout_passes=False),
    out_type=[
        jax.ShapeDtypeStruct((NW * 64,), jnp.float32),
        jax.ShapeDtypeStruct((NW * 32,), jnp.float32),
    ],
    scratch_types=[
        pltpu.VMEM((D, CH), jnp.float32),
        pltpu.VMEM((D, CH), jnp.float32),
        pltpu.VMEM((KEYS,), jnp.float32),
        pltpu.VMEM((KEYS,), jnp.int32),
        pltpu.VMEM((64,), jnp.float32),
        pltpu.VMEM((16,), jnp.float32),
        pltpu.VMEM((1024,), jnp.float32),
        pltpu.VMEM((1024,), jnp.float32),
        pltpu.VMEM((64,), jnp.float32),
        pltpu.VMEM((32,), jnp.float32),
        pltpu.SemaphoreType.DMA,
        pltpu.SemaphoreType.DMA,
    ],
)
def _sc_attn(q_hbm, kt_hbm, vt_hbm, pos_hbm, qp_hbm, acc_out, st_out,
             buf0, buf1, scores, posb, qv, qpb, qb, accv, stacc, ststat,
             sem0, sem1):
    cid = lax.axis_index("c")
    sid = lax.axis_index("s")
    wid = cid * 16 + sid
    b = wid // SEG
    seg = wid % SEG
    base = MS + seg * KEYS

    pltpu.sync_copy(q_hbm.at[b], qv)
    pltpu.sync_copy(qp_hbm, qpb)
    pltpu.sync_copy(pos_hbm.at[b, pl.ds(base, KEYS)], posb)

    lane = lax.iota(jnp.int32, 16)
    zero = jnp.zeros((16,), jnp.float32)
    qp_vec = qpb[...]

    for i in range(4):
        qvec = _bf16_rne(qv[pl.ds(16 * i, 16)])
        for j in range(16):
            qb[pl.ds((16 * i + j) * 16, 16)] = zero + qvec[j]
            accv[pl.ds((16 * i + j) * 16, 16)] = zero

    def kcopy(c, buf, sem):
        return pltpu.make_async_copy(
            kt_hbm.at[b, :, pl.ds(base + c * CH, CH)], buf, sem)

    def vcopy(c, buf, sem):
        return pltpu.make_async_copy(
            vt_hbm.at[b, :, pl.ds(base + c * CH, CH)], buf, sem)

    def p1_chunk(c, buf, m_run):
        def block(bi, m_run):
            k0 = bi * 64

            def dloop(d, svs):
                s0, s1, s2, s3 = svs
                qd = qb[pl.ds(d * 16, 16)]
                s0 = s0 + qd * _bf16_rne(buf[d, pl.ds(k0, 16)])
                s1 = s1 + qd * _bf16_rne(buf[d, pl.ds(k0 + 16, 16)])
                s2 = s2 + qd * _bf16_rne(buf[d, pl.ds(k0 + 32, 16)])
                s3 = s3 + qd * _bf16_rne(buf[d, pl.ds(k0 + 48, 16)])
                return (s0, s1, s2, s3)

            svs = lax.fori_loop(0, 64, dloop, (zero, zero, zero, zero))
            kk = c * CH + k0
            for t in range(4):
                pos16 = posb[pl.ds(kk + t * 16, 16)].astype(jnp.float32)
                sv = svs[t] * SCALE - SLOPE * jnp.abs(qp_vec - pos16)
                scores[pl.ds(kk + t * 16, 16)] = sv
                m_run = jnp.maximum(m_run, sv)
            return m_run
        return lax.fori_loop(0, CH // 64, block, m_run)

    kcopy(0, buf0, sem0).start()

    def outer1(i, m_run):
        c0 = 2 * i
        kcopy(c0, buf0, sem0).wait()
        kcopy(c0 + 1, buf1, sem1).start()
        m_run = p1_chunk(c0, buf0, m_run)
        kcopy(c0 + 1, buf1, sem1).wait()

        @pl.when(i < NCH // 2 - 1)
        def _():
            kcopy(c0 + 2, buf0, sem0).start()

        m_run = p1_chunk(c0 + 1, buf1, m_run)
        return m_run

    m_run = lax.fori_loop(0, NCH // 2, outer1,
                          jnp.full((16,), -3e38, jnp.float32))
    gmax = jnp.max(m_run)

    def p2_chunk(c, buf, ws):
        def block(bi, ws):
            k0 = bi * 64
            kk = c * CH + k0
            w0 = _exp_precise(scores[pl.ds(kk, 16)] - gmax)
            w1 = _exp_precise(scores[pl.ds(kk + 16, 16)] - gmax)
            w2 = _exp_precise(scores[pl.ds(kk + 32, 16)] - gmax)
            w3 = _exp_precise(scores[pl.ds(kk + 48, 16)] - gmax)
            ws = ws + ((w0 + w1) + (w2 + w3))

            def dloop(d, t):
                acc = accv[pl.ds(d * 16, 16)]
                acc = acc + w0 * buf[d, pl.ds(k0, 16)]
                acc = acc + w1 * buf[d, pl.ds(k0 + 16, 16)]
                acc = acc + w2 * buf[d, pl.ds(k0 + 32, 16)]
                acc = acc + w3 * buf[d, pl.ds(k0 + 48, 16)]
                accv[pl.ds(d * 16, 16)] = acc
                return t

            lax.fori_loop(0, 64, dloop, 0)
            return ws
        return lax.fori_loop(0, CH // 64, block, ws)

    vcopy(0, buf0, sem0).start()

    def outer2(i, ws):
        c0 = 2 * i
        vcopy(c0, buf0, sem0).wait()
        vcopy(c0 + 1, buf1, sem1).start()
        ws = p2_chunk(c0, buf0, ws)
        vcopy(c0 + 1, buf1, sem1).wait()

        @pl.when(i < NCH // 2 - 1)
        def _():
            vcopy(c0 + 2, buf0, sem0).start()

        ws = p2_chunk(c0 + 1, buf1, ws)
        return ws

    ws = lax.fori_loop(0, NCH // 2, outer2, zero)

    for t in range(4):
        vec = zero
        for j in range(16):
            d = t * 16 + j
            vec = jnp.where(lane == j, jnp.sum(accv[pl.ds(d * 16, 16)]), vec)
        stacc[pl.ds(t * 16, 16)] = vec
    pltpu.sync_copy(stacc, acc_out.at[pl.ds(wid * 64, 64)])

    ststat[pl.ds(0, 16)] = zero + gmax
    ststat[pl.ds(16, 16)] = ws
    pltpu.sync_copy(ststat, st_out.at[pl.ds(wid * 32, 32)])


def _tc_body(q_ref, kt_ref, vt_ref, pos_ref, qp_ref, acc_out, st_out):
    qp = qp_ref[0]
    q16 = q_ref[0].astype(jnp.bfloat16)
    k16 = kt_ref[0].astype(jnp.bfloat16)
    s = lax.dot_general(q16, k16, (((1,), (0,)), ((), ())),
                        preferred_element_type=jnp.float32)
    pos = pos_ref[0].astype(jnp.float32)
    s = s * SCALE - SLOPE * jnp.abs(qp - pos)
    m_c = jnp.max(s)
    p = jnp.exp(s - m_c)
    l_c = jnp.sum(p)
    pv = lax.dot_general(p.astype(jnp.bfloat16),
                         vt_ref[0].astype(jnp.bfloat16),
                         (((1,), (1,)), ((), ())),
                         preferred_element_type=jnp.float32)
    acc_out[...] = pv.reshape(1, 1, 1, 64)
    iota = lax.broadcasted_iota(jnp.int32, (1, 1, 1, 128), 3)
    st_out[...] = jnp.where(iota == 0, m_c,
                            jnp.where(iota == 1, l_c, 0.0))


_tc_attn = pl.pallas_call(
    _tc_body,
    grid=(B, NCT if NCT else 1),
    in_specs=[
        pl.BlockSpec((1, 1, 64), lambda b, c: (b, 0, 0)),
        pl.BlockSpec((1, 64, CHT), lambda b, c: (b, 0, c)),
        pl.BlockSpec((1, 64, CHT), lambda b, c: (b, 0, c)),
        pl.BlockSpec((1, 1, CHT), lambda b, c: (b * (NCT if NCT else 1) + c, 0, 0)),
        pl.BlockSpec(memory_space=pltpu.MemorySpace.SMEM),
    ],
    out_specs=[
        pl.BlockSpec((1, 1, 1, 64), lambda b, c: (b, c, 0, 0)),
        pl.BlockSpec((1, 1, 1, 128), lambda b, c: (b, c, 0, 0)),
    ],
    out_shape=[
        jax.ShapeDtypeStruct((B, NCT if NCT else 1, 1, 64), jnp.float32),
        jax.ShapeDtypeStruct((B, NCT if NCT else 1, 1, 128), jnp.float32),
    ],
)


def kernel(query_addr, key_addrs, values, query_pos, key_pos):
    pos32 = key_pos.astype(jnp.int32)
    qp_s = jnp.full((1,), query_pos, dtype=jnp.float32)
    kt = jnp.swapaxes(key_addrs, 1, 2)
    vt = jnp.swapaxes(values, 1, 2)

    parts_m = []
    parts_l = []
    parts_acc = []

    if MSC:
        qp = jnp.full((16,), query_pos, dtype=jnp.float32)
        accf, stf = _sc_attn(query_addr, kt, vt, pos32, qp)
        acc = accf.reshape(B, SEG, 64)
        st = stf.reshape(B, SEG, 32)
        parts_m.append(st[:, :, 0])
        parts_l.append(st[:, :, 16:32].sum(-1))
        parts_acc.append(acc)

    if NCT:
        q3d = query_addr.reshape(B, 1, 64)
        pos_tc = pos32[:, :MS].reshape(B * NCT, 1, CHT)
        tacc, tst = _tc_attn(q3d, kt, vt, pos_tc, qp_s)
        parts_m.append(tst[:, :, 0, 0])
        parts_l.append(tst[:, :, 0, 1])
        parts_acc.append(tacc[:, :, 0, :])

    pm = jnp.concatenate(parts_m, axis=1)
    pl_ = jnp.concatenate(parts_l, axis=1)
    pa = jnp.concatenate(parts_acc, axis=1)
    gm = pm.max(axis=1, keepdims=True)
    sc = jnp.exp(pm - gm)
    num = (sc[:, :, None] * pa).sum(1)
    den = (sc * pl_).sum(1)[:, None]
    return num / den


# speedup vs baseline: 1.2780x; 1.0094x over previous
import functools

import jax
import jax.numpy as jnp
from jax import lax
from jax.experimental import pallas as pl
from jax.experimental.pallas import tpu as pltpu
from jax.experimental.pallas import tpu_sc as plsc

B = 8
M = 65536
D = 64
SCALE = 2.5
SLOPE = 0.01

MSC = 24576
MS = M - MSC

SEG = 4
NW = 32
KEYS = max(MSC // SEG, 1024)
CH = 512
NCH = KEYS // CH
GR = CH // 16

CHT = 4096
NCT = MS // CHT

_LOG2E = 1.4426950408889634
_LN2 = 0.6931471805599453
_RND = 12582912.0


def _bf16_rne(x):
    i = lax.bitcast_convert_type(x, jnp.int32)
    lsb = lax.shift_right_logical(i, 16) & 1
    r = (i + 0x7FFF + lsb) & jnp.int32(-65536)
    return lax.bitcast_convert_type(r, jnp.float32)


def _exp_precise(x):
    x = jnp.maximum(x, -87.0)
    t = x * _LOG2E
    n_f = (t + _RND) - _RND
    r = (t - n_f) * _LN2
    p = 1.0 + r * (1.0 + r * (0.5 + r * (1.0 / 6.0 + r * (
        1.0 / 24.0 + r * (1.0 / 120.0 + r * (1.0 / 720.0))))))
    n_i = n_f.astype(jnp.int32)
    scale = lax.bitcast_convert_type((n_i + 127) << 23, jnp.float32)
    return p * scale


_mesh = plsc.VectorSubcoreMesh(core_axis_name="c", subcore_axis_name="s")


@functools.partial(
    pl.kernel,
    mesh=_mesh,
    compiler_params=pltpu.CompilerParams(needs_layout_passes=False),
    out_type=[
        jax.ShapeDtypeStruct((NW * 64,), jnp.float32),
        jax.ShapeDtypeStruct((NW * 32,), jnp.float32),
    ],
    scratch_types=[
        pltpu.VMEM((D, CH), jnp.float32),
        pltpu.VMEM((D, CH), jnp.float32),
        pltpu.VMEM((KEYS,), jnp.float32),
        pltpu.VMEM((KEYS,), jnp.int32),
        pltpu.VMEM((64,), jnp.float32),
        pltpu.VMEM((16,), jnp.float32),
        pltpu.VMEM((1024,), jnp.float32),
        pltpu.VMEM((1024,), jnp.float32),
        pltpu.VMEM((64,), jnp.float32),
        pltpu.VMEM((32,), jnp.float32),
        pltpu.SemaphoreType.DMA,
        pltpu.SemaphoreType.DMA,
    ],
)
def _sc_attn(q_hbm, kt_hbm, vt_hbm, pos_hbm, qp_hbm, acc_out, st_out,
             buf0, buf1, scores, posb, qv, qpb, qb, accv, stacc, ststat,
             sem0, sem1):
    cid = lax.axis_index("c")
    sid = lax.axis_index("s")
    wid = cid * 16 + sid
    b = wid // SEG
    seg = wid % SEG
    base = MS + seg * KEYS

    pltpu.sync_copy(q_hbm.at[b], qv)
    pltpu.sync_copy(qp_hbm, qpb)
    pltpu.sync_copy(pos_hbm.at[b, pl.ds(base, KEYS)], posb)

    lane = lax.iota(jnp.int32, 16)
    zero = jnp.zeros((16,), jnp.float32)
    qp_vec = qpb[...]

    for i in range(4):
        qvec = _bf16_rne(qv[pl.ds(16 * i, 16)])
        for j in range(16):
            qb[pl.ds((16 * i + j) * 16, 16)] = zero + qvec[j]
            accv[pl.ds((16 * i + j) * 16, 16)] = zero

    def kcopy(c, buf, sem):
        return pltpu.make_async_copy(
            kt_hbm.at[b, :, pl.ds(base + c * CH, CH)], buf, sem)

    def vcopy(c, buf, sem):
        return pltpu.make_async_copy(
            vt_hbm.at[b, :, pl.ds(base + c * CH, CH)], buf, sem)

    def p1_chunk(c, buf, m_run):
        def block(bi, m_run):
            k0 = bi * 64

            def dloop(d, svs):
                s0, s1, s2, s3 = svs
                qd = qb[pl.ds(d * 16, 16)]
                s0 = s0 + qd * _bf16_rne(buf[d, pl.ds(k0, 16)])
                s1 = s1 + qd * _bf16_rne(buf[d, pl.ds(k0 + 16, 16)])
                s2 = s2 + qd * _bf16_rne(buf[d, pl.ds(k0 + 32, 16)])
                s3 = s3 + qd * _bf16_rne(buf[d, pl.ds(k0 + 48, 16)])
                return (s0, s1, s2, s3)

            svs = lax.fori_loop(0, 64, dloop, (zero, zero, zero, zero))
            kk = c * CH + k0
            for t in range(4):
                pos16 = posb[pl.ds(kk + t * 16, 16)].astype(jnp.float32)
                sv = svs[t] * SCALE - SLOPE * jnp.abs(qp_vec - pos16)
                scores[pl.ds(kk + t * 16, 16)] = sv
                m_run = jnp.maximum(m_run, sv)
            return m_run
        return lax.fori_loop(0, CH // 64, block, m_run)

    kcopy(0, buf0, sem0).start()

    def outer1(i, m_run):
        c0 = 2 * i
        kcopy(c0, buf0, sem0).wait()
        kcopy(c0 + 1, buf1, sem1).start()
        m_run = p1_chunk(c0, buf0, m_run)
        kcopy(c0 + 1, buf1, sem1).wait()

        @pl.when(i < NCH // 2 - 1)
        def _():
            kcopy(c0 + 2, buf0, sem0).start()

        m_run = p1_chunk(c0 + 1, buf1, m_run)
        return m_run

    m_run = lax.fori_loop(0, NCH // 2, outer1,
                          jnp.full((16,), -3e38, jnp.float32))
    gmax = jnp.max(m_run)

    def p2_chunk(c, buf, ws):
        def block(bi, ws):
            k0 = bi * 64
            kk = c * CH + k0
            w0 = _exp_precise(scores[pl.ds(kk, 16)] - gmax)
            w1 = _exp_precise(scores[pl.ds(kk + 16, 16)] - gmax)
            w2 = _exp_precise(scores[pl.ds(kk + 32, 16)] - gmax)
            w3 = _exp_precise(scores[pl.ds(kk + 48, 16)] - gmax)
            ws = ws + ((w0 + w1) + (w2 + w3))

            def dloop(d, t):
                acc = accv[pl.ds(d * 16, 16)]
                acc = acc + w0 * buf[d, pl.ds(k0, 16)]
                acc = acc + w1 * buf[d, pl.ds(k0 + 16, 16)]
                acc = acc + w2 * buf[d, pl.ds(k0 + 32, 16)]
                acc = acc + w3 * buf[d, pl.ds(k0 + 48, 16)]
                accv[pl.ds(d * 16, 16)] = acc
                return t

            lax.fori_loop(0, 64, dloop, 0)
            return ws
        return lax.fori_loop(0, CH // 64, block, ws)

    vcopy(0, buf0, sem0).start()

    def outer2(i, ws):
        c0 = 2 * i
        vcopy(c0, buf0, sem0).wait()
        vcopy(c0 + 1, buf1, sem1).start()
        ws = p2_chunk(c0, buf0, ws)
        vcopy(c0 + 1, buf1, sem1).wait()

        @pl.when(i < NCH // 2 - 1)
        def _():
            vcopy(c0 + 2, buf0, sem0).start()

        ws = p2_chunk(c0 + 1, buf1, ws)
        return ws

    ws = lax.fori_loop(0, NCH // 2, outer2, zero)

    for t in range(4):
        vec = zero
        for j in range(16):
            d = t * 16 + j
            vec = jnp.where(lane == j, jnp.sum(accv[pl.ds(d * 16, 16)]), vec)
        stacc[pl.ds(t * 16, 16)] = vec
    pltpu.sync_copy(stacc, acc_out.at[pl.ds(wid * 64, 64)])

    ststat[pl.ds(0, 16)] = zero + gmax
    ststat[pl.ds(16, 16)] = ws
    pltpu.sync_copy(ststat, st_out.at[pl.ds(wid * 32, 32)])


def _tc_body(q_ref, kt_ref, vt_ref, pos_ref, qp_ref, acc_out, st_out):
    qp = qp_ref[0]
    q16 = q_ref[0].astype(jnp.bfloat16)
    k16 = kt_ref[0].astype(jnp.bfloat16)
    s = lax.dot_general(q16, k16, (((1,), (0,)), ((), ())),
                        preferred_element_type=jnp.float32)
    pos = pos_ref[0].astype(jnp.float32)
    s = s * SCALE - SLOPE * jnp.abs(qp - pos)
    m_c = jnp.max(s)
    p = jnp.exp(s - m_c)
    l_c = jnp.sum(p)
    pv = lax.dot_general(p.astype(jnp.bfloat16),
                         vt_ref[0].astype(jnp.bfloat16),
                         (((1,), (1,)), ((), ())),
                         preferred_element_type=jnp.float32)
    acc_out[...] = pv.reshape(1, 1, 1, 64)
    iota = lax.broadcasted_iota(jnp.int32, (1, 1, 1, 128), 3)
    st_out[...] = jnp.where(iota == 0, m_c,
                            jnp.where(iota == 1, l_c, 0.0))


_tc_attn = pl.pallas_call(
    _tc_body,
    grid=(B, NCT if NCT else 1),
    in_specs=[
        pl.BlockSpec((1, 1, 64), lambda b, c: (b, 0, 0)),
        pl.BlockSpec((1, 64, CHT), lambda b, c: (b, 0, c)),
        pl.BlockSpec((1, 64, CHT), lambda b, c: (b, 0, c)),
        pl.BlockSpec((1, 1, CHT), lambda b, c: (b * (NCT if NCT else 1) + c, 0, 0)),
        pl.BlockSpec(memory_space=pltpu.MemorySpace.SMEM),
    ],
    out_specs=[
        pl.BlockSpec((1, 1, 1, 64), lambda b, c: (b, c, 0, 0)),
        pl.BlockSpec((1, 1, 1, 128), lambda b, c: (b, c, 0, 0)),
    ],
    out_shape=[
        jax.ShapeDtypeStruct((B, NCT if NCT else 1, 1, 64), jnp.float32),
        jax.ShapeDtypeStruct((B, NCT if NCT else 1, 1, 128), jnp.float32),
    ],
)


def kernel(query_addr, key_addrs, values, query_pos, key_pos):
    pos32 = key_pos.astype(jnp.int32)
    qp_s = jnp.full((1,), query_pos, dtype=jnp.float32)
    kt = jnp.swapaxes(key_addrs, 1, 2)
    vt = jnp.swapaxes(values, 1, 2)

    parts_m = []
    parts_l = []
    parts_acc = []

    if MSC:
        qp = jnp.full((16,), query_pos, dtype=jnp.float32)
        accf, stf = _sc_attn(query_addr, kt, vt, pos32, qp)
        acc = accf.reshape(B, SEG, 64)
        st = stf.reshape(B, SEG, 32)
        parts_m.append(st[:, :, 0])
        parts_l.append(st[:, :, 16:32].sum(-1))
        parts_acc.append(acc)

    if NCT:
        q3d = query_addr.reshape(B, 1, 64)
        pos_tc = pos32[:, :MS].reshape(B * NCT, 1, CHT)
        tacc, tst = _tc_attn(q3d, kt, vt, pos_tc, qp_s)
        parts_m.append(tst[:, :, 0, 0])
        parts_l.append(tst[:, :, 0, 1])
        parts_acc.append(tacc[:, :, 0, :])

    pm = jnp.concatenate(parts_m, axis=1)
    pl_ = jnp.concatenate(parts_l, axis=1)
    pa = jnp.concatenate(parts_acc, axis=1)
    gm = pm.max(axis=1, keepdims=True)
    sc = jnp.exp(pm - gm)
    num = (sc[:, :, None] * pa).sum(1)
    den = (sc * pl_).sum(1)[:, None]
    return num / den
